# baseline (speedup 1.0000x reference)
import math

import jax
import jax.numpy as jnp
from jax.experimental import pallas as pl
from jax.experimental.pallas import tpu as pltpu

N_BUCKETS = 32
MAX_DISTANCE = 128
N_HEAD = 16
Q_LEN = 2048
K_LEN = 2048

BQ = 512
D8_W = 4224
D128_W = 4096


def _bias_block_kernel(delta_ref, wt_ref, out_ref, d8_ref, d128_ref):
    qb = pl.program_id(1)
    delta = delta_ref[0]

    @pl.when(qb == 0)
    def _build_tables():
        i = jax.lax.broadcasted_iota(jnp.int32, (8, D8_W), 0)
        u = jax.lax.broadcasted_iota(jnp.int32, (8, D8_W), 1)
        rel = (u - i) - (Q_LEN - 1) + delta

        half = N_BUCKETS // 2
        bucket = jnp.where(rel > 0, half, 0).astype(jnp.int32)
        arel = jnp.abs(rel)
        max_exact = half // 2
        is_small = arel < max_exact
        me = max_exact - 1
        nb = half - 1
        safe = jnp.maximum(arel.astype(jnp.float32), 1.0)
        rp_large = me + (
            jnp.log(safe / me) / math.log(MAX_DISTANCE / me) * (nb - me)
        ).astype(jnp.int32)
        rp_large = jnp.minimum(rp_large, nb)
        bucket = bucket + jnp.where(is_small, arel, rp_large)

        w_row = wt_ref[0, 0, :]
        vals = jnp.zeros((8, D8_W), jnp.float32)
        for b in range(N_BUCKETS):
            vals = jnp.where(bucket == b, w_row[b], vals)
        d8_ref[:, :] = vals

        for m in range(16):
            d128_ref[pl.ds(8 * m, 8), :] = d8_ref[:, pl.ds(127 - 8 * m, D128_W)]

    q0 = qb * BQ
    for p in range(BQ // 128):
        s = (Q_LEN - 128) - q0 - 128 * p
        out_ref[0, 0, pl.ds(128 * p, 128), :] = d128_ref[:, pl.ds(s, K_LEN)]


def _run_heads(delta, wt, n_heads):
    return pl.pallas_call(
        _bias_block_kernel,
        grid=(n_heads, Q_LEN // BQ),
        in_specs=[
            pl.BlockSpec(memory_space=pltpu.SMEM),
            pl.BlockSpec((1, 1, N_BUCKETS), lambda h, qb: (h, 0, 0)),
        ],
        out_specs=pl.BlockSpec((1, 1, BQ, K_LEN), lambda h, qb: (0, h, qb, 0)),
        out_shape=jax.ShapeDtypeStruct((1, n_heads, Q_LEN, K_LEN), jnp.float32),
        scratch_shapes=[
            pltpu.VMEM((8, D8_W), jnp.float32),
            pltpu.VMEM((128, D128_W), jnp.float32),
        ],
        compiler_params=pltpu.CompilerParams(
            dimension_semantics=("parallel", "arbitrary"),
        ),
    )(delta, wt)


def kernel(relative_bias_weight, query_length, key_length):
    delta = (
        jnp.asarray(key_length, jnp.int32) - K_LEN
        - (jnp.asarray(query_length, jnp.int32) - Q_LEN)
    ).reshape(1)
    wt = relative_bias_weight.T.reshape(N_HEAD, 1, N_BUCKETS)

    h_split = 8
    out_a = _run_heads(delta, wt[:h_split], h_split)
    out_b = _run_heads(delta, wt[h_split:], N_HEAD - h_split)
    return jnp.concatenate([out_a, out_b], axis=1)


# speedup vs baseline: 2.3709x; 2.3709x over previous
import jax
import jax.numpy as jnp
from jax import lax
from jax.experimental import pallas as pl
from jax.experimental.pallas import tpu as pltpu
from jax.experimental.pallas import tpu_sc as plsc

N_BUCKETS = 32
N_HEAD = 16
Q_LEN = 2048
K_LEN = 2048

BQ = 512
DIAG_N = 4096
D8_W = 4224
D128_W = 4096

_THRESHOLDS = (11, 15, 21, 30, 44, 62, 90, 128)
_HALF_DIAG = DIAG_N // 2


def _sc_diag_kernel(wt_hbm, delta_hbm, out_hbm, w_v, d_v, row_v):
    h = lax.axis_index("s")
    half = lax.axis_index("c")
    pltpu.sync_copy(wt_hbm.at[h], w_v)
    pltpu.sync_copy(delta_hbm, d_v)
    dvec = d_v[...]
    w_lo = w_v[pl.ds(0, 16)]
    w_hi = w_v[pl.ds(16, 16)]
    base = half * _HALF_DIAG

    def body(i, carry):
        t = base + i * 16 + lax.broadcasted_iota(jnp.int32, (16,), 0)
        rel = t - (Q_LEN - 1) + dvec
        a = jnp.abs(rel)
        b = jnp.minimum(a, 7)
        for thr in _THRESHOLDS:
            b = b + jnp.where(a >= thr, 1, 0)
        lo = w_lo.at[b].get(mode="promise_in_bounds")
        hi = w_hi.at[b].get(mode="promise_in_bounds")
        row_v[pl.ds(i * 16, 16)] = jnp.where(rel > 0, hi, lo)
        return carry

    lax.fori_loop(0, _HALF_DIAG // 16, body, 0)
    pltpu.sync_copy(row_v, out_hbm.at[h, pl.ds(base, _HALF_DIAG)])


def _tc_expand_kernel(diag_ref, out_ref, d8_ref, d128_ref):
    qb = pl.program_id(1)

    @pl.when(qb == 0)
    def _build_tables():
        row = diag_ref[0, 0, :].reshape(1, DIAG_N)
        for i in range(8):
            d8_ref[pl.ds(i, 1), pl.ds(i, DIAG_N)] = row
        for m in range(16):
            d128_ref[pl.ds(8 * m, 8), :] = d8_ref[:, pl.ds(127 - 8 * m, D128_W)]

    q0 = qb * BQ
    for p in range(BQ // 128):
        s = (Q_LEN - 128) - q0 - 128 * p
        out_ref[0, 0, pl.ds(128 * p, 128), :] = d128_ref[:, pl.ds(s, K_LEN)]


def kernel(relative_bias_weight, query_length, key_length):
    delta = (
        jnp.asarray(key_length, jnp.int32) - K_LEN
        - (jnp.asarray(query_length, jnp.int32) - Q_LEN)
    )
    wt = relative_bias_weight.T
    delta_vec = jnp.full((16,), delta, jnp.int32)

    diag = pl.kernel(
        _sc_diag_kernel,
        out_type=jax.ShapeDtypeStruct((N_HEAD, DIAG_N), jnp.float32),
        mesh=plsc.VectorSubcoreMesh(core_axis_name="c", subcore_axis_name="s"),
        scratch_types=[
            pltpu.VMEM((N_BUCKETS,), jnp.float32),
            pltpu.VMEM((16,), jnp.int32),
            pltpu.VMEM((_HALF_DIAG,), jnp.float32),
        ],
    )(wt, delta_vec)

    out = pl.pallas_call(
        _tc_expand_kernel,
        grid=(N_HEAD, Q_LEN // BQ),
        in_specs=[
            pl.BlockSpec((1, 1, DIAG_N), lambda h, qb: (h, 0, 0)),
        ],
        out_specs=pl.BlockSpec((1, 1, BQ, K_LEN), lambda h, qb: (0, h, qb, 0)),
        out_shape=jax.ShapeDtypeStruct((1, N_HEAD, Q_LEN, K_LEN), jnp.float32),
        scratch_shapes=[
            pltpu.VMEM((8, D8_W), jnp.float32),
            pltpu.VMEM((128, D128_W), jnp.float32),
        ],
        compiler_params=pltpu.CompilerParams(
            dimension_semantics=("parallel", "arbitrary"),
        ),
    )(diag.reshape(N_HEAD, 1, DIAG_N))
    return out


# speedup vs baseline: 2.5200x; 1.0629x over previous
import jax
import jax.numpy as jnp
from jax import lax
from jax.experimental import pallas as pl
from jax.experimental.pallas import tpu as pltpu
from jax.experimental.pallas import tpu_sc as plsc

N_BUCKETS = 32
N_HEAD = 16
Q_LEN = 2048
K_LEN = 2048

BQ = 1024
DIAG_N = 4096
D8_W = 4224
D128_W = 4096

_THRESHOLDS = (11, 15, 21, 30, 44, 62, 90, 128)
_HALF_DIAG = DIAG_N // 2


def _sc_diag_kernel(wt_hbm, delta_hbm, out_hbm, w_v, d_v, row_v):
    h = lax.axis_index("s")
    half = lax.axis_index("c")
    pltpu.sync_copy(wt_hbm.at[h], w_v)
    pltpu.sync_copy(delta_hbm, d_v)
    dvec = d_v[...]
    w_lo = w_v[pl.ds(0, 16)]
    w_hi = w_v[pl.ds(16, 16)]
    base = half * _HALF_DIAG

    def body(i, carry):
        t = base + i * 16 + lax.broadcasted_iota(jnp.int32, (16,), 0)
        rel = t - (Q_LEN - 1) + dvec
        a = jnp.abs(rel)
        b = jnp.minimum(a, 7)
        for thr in _THRESHOLDS:
            b = b + jnp.where(a >= thr, 1, 0)
        lo = w_lo.at[b].get(mode="promise_in_bounds")
        hi = w_hi.at[b].get(mode="promise_in_bounds")
        row_v[pl.ds(i * 16, 16)] = jnp.where(rel > 0, hi, lo)
        return carry

    lax.fori_loop(0, _HALF_DIAG // 16, body, 0)
    pltpu.sync_copy(row_v, out_hbm.at[h, pl.ds(base, _HALF_DIAG)])


def _tc_expand_kernel(diag_ref, out_ref, d8_ref, d128_ref):
    qb = pl.program_id(1)

    @pl.when(qb == 0)
    def _build_tables():
        row = diag_ref[0, 0, :].reshape(1, DIAG_N)
        for i in range(8):
            d8_ref[pl.ds(i, 1), pl.ds(i, DIAG_N)] = row
        for m in range(16):
            d128_ref[pl.ds(8 * m, 8), :] = d8_ref[:, pl.ds(127 - 8 * m, D128_W)]

    q0 = qb * BQ
    for p in range(BQ // 128):
        s = (Q_LEN - 128) - q0 - 128 * p
        out_ref[0, 0, pl.ds(128 * p, 128), :] = d128_ref[:, pl.ds(s, K_LEN)]


def kernel(relative_bias_weight, query_length, key_length):
    delta = (
        jnp.asarray(key_length, jnp.int32) - K_LEN
        - (jnp.asarray(query_length, jnp.int32) - Q_LEN)
    )
    wt = relative_bias_weight.T
    delta_vec = jnp.full((16,), delta, jnp.int32)

    diag = pl.kernel(
        _sc_diag_kernel,
        out_type=jax.ShapeDtypeStruct((N_HEAD, DIAG_N), jnp.float32),
        mesh=plsc.VectorSubcoreMesh(core_axis_name="c", subcore_axis_name="s"),
        scratch_types=[
            pltpu.VMEM((N_BUCKETS,), jnp.float32),
            pltpu.VMEM((16,), jnp.int32),
            pltpu.VMEM((_HALF_DIAG,), jnp.float32),
        ],
    )(wt, delta_vec)

    out = pl.pallas_call(
        _tc_expand_kernel,
        grid=(N_HEAD, Q_LEN // BQ),
        in_specs=[
            pl.BlockSpec((1, 1, DIAG_N), lambda h, qb: (h, 0, 0)),
        ],
        out_specs=pl.BlockSpec((1, 1, BQ, K_LEN), lambda h, qb: (0, h, qb, 0)),
        out_shape=jax.ShapeDtypeStruct((1, N_HEAD, Q_LEN, K_LEN), jnp.float32),
        scratch_shapes=[
            pltpu.VMEM((8, D8_W), jnp.float32),
            pltpu.VMEM((128, D128_W), jnp.float32),
        ],
        compiler_params=pltpu.CompilerParams(
            dimension_semantics=("parallel", "arbitrary"),
        ),
    )(diag.reshape(N_HEAD, 1, DIAG_N))
    return out
